# Initial kernel scaffold; baseline (speedup 1.0000x reference)
import jax
import jax.numpy as jnp
from jax.experimental import pallas as pl


def kernel(x, gate_w, w1, w2, w3):
    raise NotImplementedError("write your pallas kernel here")


# speedup vs baseline: 3.8490x; 3.8490x over previous
import functools

import jax
import jax.numpy as jnp
from jax.experimental import pallas as pl
from jax.experimental.pallas import tpu as pltpu

N_TOKENS = 2048
DIM = 1024
HIDDEN = 1024
NUM_EXPERTS = 8
TOP_K = 2
EPAD = 128


def _moe_body(x_ref, gwp_ref, w1_ref, w3_ref, w2_ref, out_ref, wts_ref):
    e = pl.program_id(1)

    @pl.when(e == 0)
    def _router():
        x = x_ref[...]
        logits = jax.lax.dot_general(
            x, gwp_ref[...], (((1,), (1,)), ((), ())),
            preferred_element_type=jnp.float32)
        lane = jax.lax.broadcasted_iota(jnp.int32, logits.shape, 1)
        neg = jnp.float32(-1e30)
        logits = jnp.where(lane < NUM_EXPERTS, logits, neg)
        a1 = jnp.argmax(logits, axis=1)[:, None]
        l2 = jnp.where(lane == a1, neg, logits)
        a2 = jnp.argmax(l2, axis=1)[:, None]
        sel = (lane == a1) | (lane == a2)
        scores = jax.nn.sigmoid(logits)
        wts_ref[...] = jnp.where(sel, scores, 0.0)
        out_ref[...] = jnp.zeros_like(out_ref)

    wsel = jnp.sum(
        jnp.where(
            jax.lax.broadcasted_iota(jnp.int32, wts_ref.shape, 1) == e,
            wts_ref[...], 0.0),
        axis=1, keepdims=True)
    xm = x_ref[...] * wsel
    h1 = jax.lax.dot_general(xm, w1_ref[0], (((1,), (1,)), ((), ())),
                             preferred_element_type=jnp.float32)
    h3 = jax.lax.dot_general(xm, w3_ref[0], (((1,), (1,)), ((), ())),
                             preferred_element_type=jnp.float32)
    h = jax.nn.silu(h1) * h3
    out_ref[...] += jax.lax.dot_general(h, w2_ref[0], (((1,), (1,)), ((), ())),
                                        preferred_element_type=jnp.float32)


@jax.jit
def _moe(x2d, gwp, w1, w2, w3, *, bt=1024):
    nt = x2d.shape[0] // bt
    return pl.pallas_call(
        _moe_body,
        grid=(nt, NUM_EXPERTS),
        in_specs=[
            pl.BlockSpec((bt, DIM), lambda t, e: (t, 0)),
            pl.BlockSpec((EPAD, DIM), lambda t, e: (0, 0)),
            pl.BlockSpec((1, HIDDEN, DIM), lambda t, e: (e, 0, 0)),
            pl.BlockSpec((1, HIDDEN, DIM), lambda t, e: (e, 0, 0)),
            pl.BlockSpec((1, DIM, HIDDEN), lambda t, e: (e, 0, 0)),
        ],
        out_specs=pl.BlockSpec((bt, DIM), lambda t, e: (t, 0)),
        out_shape=jax.ShapeDtypeStruct((x2d.shape[0], DIM), jnp.float32),
        scratch_shapes=[pltpu.VMEM((bt, EPAD), jnp.float32)],
    )(x2d, gwp, w1, w3, w2)


def kernel(x, gate_w, w1, w2, w3):
    orig_shape = x.shape
    x2d = x.reshape(-1, orig_shape[-1])
    gwp = jnp.zeros((EPAD, DIM), jnp.float32).at[:NUM_EXPERTS].set(gate_w)
    out = _moe(x2d, gwp, w1, w2, w3)
    return out.reshape(orig_shape)
